# baseline (speedup 1.0000x reference)
import functools
import jax
import jax.numpy as jnp
from jax import lax
from jax.experimental import pallas as pl
from jax.experimental.pallas import tpu as pltpu
from jax.experimental.pallas import tpu_sc as plsc

C = 128
G = 64
S = G * G * G
NC = 2
NS = 16
NW = NC * NS
B = 16
NP = 102400


def _tr_body(g_ref, t_ref):
    t_ref[...] = g_ref[...].T


def _transpose(gflat):
    BS = 2048
    return pl.pallas_call(
        _tr_body,
        grid=(S // BS,),
        in_specs=[pl.BlockSpec((C, BS), lambda j: (0, j))],
        out_specs=pl.BlockSpec((BS, C), lambda j: (j, 0)),
        out_shape=jax.ShapeDtypeStruct((S, C), jnp.float32),
    )(gflat)


def _axis_coords(v):
    t = ((v * 2.0 - 1.0) + 1.0) * 0.5 * (G - 1.0)
    ti = t.astype(jnp.int32)
    tf = ti.astype(jnp.float32)
    ti = jnp.where(tf > t, ti - 1, ti)
    tf = ti.astype(jnp.float32)
    w = t - tf
    i0 = jnp.clip(ti, 0, G - 1)
    i1 = jnp.minimum(i0 + 1, G - 1)
    return i0, i1, w


def _sc_sample(table, vx, vy, vz, pf):
    per_w = NP // NW
    nblk = per_w // B

    mesh = plsc.VectorSubcoreMesh(core_axis_name="c", subcore_axis_name="s")

    @functools.partial(
        pl.kernel,
        out_type=jax.ShapeDtypeStruct((NP, 2 * C), jnp.float32),
        mesh=mesh,
        compiler_params=pltpu.CompilerParams(needs_layout_passes=False),
        scratch_types=[
            pltpu.VMEM((B,), jnp.float32),
            pltpu.VMEM((B,), jnp.float32),
            pltpu.VMEM((B,), jnp.float32),
            pltpu.VMEM((3 * B,), jnp.float32),
            pltpu.VMEM((8 * B,), jnp.int32),
            pltpu.VMEM((8 * B, C), jnp.float32),
            pltpu.VMEM((B, 2 * C), jnp.float32),
            pltpu.SemaphoreType.DMA,
        ],
    )
    def k(table_h, vx_h, vy_h, vz_h, pf_h, out_h,
          vxb, vyb, vzb, wbuf, idxbuf, rows, outbuf, gsem):
        wid = lax.axis_index("s") * NC + lax.axis_index("c")
        wbase = wid * per_w

        def block(i, carry):
            base = wbase + i * B
            pltpu.sync_copy(vx_h.at[pl.ds(base, B)], vxb)
            pltpu.sync_copy(vy_h.at[pl.ds(base, B)], vyb)
            pltpu.sync_copy(vz_h.at[pl.ds(base, B)], vzb)

            x0, x1, wx = _axis_coords(vxb[...])
            y0, y1, wy = _axis_coords(vyb[...])
            z0, z1, wz = _axis_coords(vzb[...])
            wbuf[pl.ds(0, B)] = wx
            wbuf[pl.ds(B, B)] = wy
            wbuf[pl.ds(2 * B, B)] = wz

            a0 = z0 * (G * G)
            a1 = z1 * (G * G)
            b0 = y0 * G
            b1 = y1 * G
            idxbuf[pl.ds(0 * B, B)] = a0 + b0 + x0
            idxbuf[pl.ds(1 * B, B)] = a0 + b0 + x1
            idxbuf[pl.ds(2 * B, B)] = a0 + b1 + x0
            idxbuf[pl.ds(3 * B, B)] = a0 + b1 + x1
            idxbuf[pl.ds(4 * B, B)] = a1 + b0 + x0
            idxbuf[pl.ds(5 * B, B)] = a1 + b0 + x1
            idxbuf[pl.ds(6 * B, B)] = a1 + b1 + x0
            idxbuf[pl.ds(7 * B, B)] = a1 + b1 + x1

            pltpu.async_copy(table_h.at[idxbuf], rows, gsem)
            pltpu.sync_copy(pf_h.at[pl.ds(base, B), :], outbuf.at[:, pl.ds(0, C)])
            pltpu.make_async_copy(table_h.at[idxbuf], rows, gsem).wait()

            def point(p, pcarry):
                pv = jnp.full((B,), p, jnp.int32)
                wxp = plsc.load_gather(wbuf, [pv])
                wyp = plsc.load_gather(wbuf, [pv + B])
                wzp = plsc.load_gather(wbuf, [pv + 2 * B])
                for cg in range(C // 16):
                    sl = pl.ds(cg * 16, 16)
                    c000 = rows[0 * B + p, sl]
                    c001 = rows[1 * B + p, sl]
                    c010 = rows[2 * B + p, sl]
                    c011 = rows[3 * B + p, sl]
                    c100 = rows[4 * B + p, sl]
                    c101 = rows[5 * B + p, sl]
                    c110 = rows[6 * B + p, sl]
                    c111 = rows[7 * B + p, sl]
                    c00 = c000 + wxp * (c001 - c000)
                    c01 = c010 + wxp * (c011 - c010)
                    c10 = c100 + wxp * (c101 - c100)
                    c11 = c110 + wxp * (c111 - c110)
                    c0 = c00 + wyp * (c01 - c00)
                    c1 = c10 + wyp * (c11 - c10)
                    outbuf[p, pl.ds(C + cg * 16, 16)] = c0 + wzp * (c1 - c0)
                return pcarry

            lax.fori_loop(0, B, point, 0)
            pltpu.sync_copy(outbuf, out_h.at[pl.ds(base, B), :])
            return carry

        lax.fori_loop(0, nblk, block, 0)

    return k(table, vx, vy, vz, pf)


def kernel(grid_batch_features, vertices, point_feat):
    grid = grid_batch_features[0].reshape(C, S)
    table = _transpose(grid)
    n = vertices.shape[0]
    pad = NP - n
    vx = jnp.pad(vertices[:, 0], (0, pad))
    vy = jnp.pad(vertices[:, 1], (0, pad))
    vz = jnp.pad(vertices[:, 2], (0, pad))
    pfp = jnp.pad(point_feat, ((0, pad), (0, 0)))
    out = _sc_sample(table, vx, vy, vz, pfp)
    return out[:n]


# speedup vs baseline: 1.8493x; 1.8493x over previous
import functools
import jax
import jax.numpy as jnp
import numpy as np
from jax import lax
from jax.experimental import pallas as pl
from jax.experimental.pallas import tpu as pltpu
from jax.experimental.pallas import tpu_sc as plsc

C = 128
G = 64
S = G * G * G
NC = 2
NS = 16
NW = NC * NS
B = 16
N = 100000
WPTS = 3136
NBLK = WPTS // B
NB = 4
STARTS = [8 * ((w * ((N - WPTS) // 8)) // (NW - 1)) for w in range(NW)]


def _tr_body(g_ref, t_ref):
    t_ref[...] = g_ref[...].T


def _transpose(gflat):
    BS = 2048
    return pl.pallas_call(
        _tr_body,
        grid=(S // BS,),
        in_specs=[pl.BlockSpec((C, BS), lambda j: (0, j))],
        out_specs=pl.BlockSpec((BS, C), lambda j: (j, 0)),
        out_shape=jax.ShapeDtypeStruct((S, C), jnp.float32),
    )(gflat)


def _axis_coords(v):
    t = ((v * 2.0 - 1.0) + 1.0) * 0.5 * (G - 1.0)
    ti = t.astype(jnp.int32)
    tf = ti.astype(jnp.float32)
    ti = jnp.where(tf > t, ti - 1, ti)
    tf = ti.astype(jnp.float32)
    w = t - tf
    i0 = jnp.clip(ti, 0, G - 1)
    i1 = jnp.minimum(i0 + 1, G - 1)
    return i0, i1, w


def _fill_block(vx, vy, vz, idxb, wb):
    x0, x1, wx = _axis_coords(vx)
    y0, y1, wy = _axis_coords(vy)
    z0, z1, wz = _axis_coords(vz)
    wb[pl.ds(0, B)] = wx
    wb[pl.ds(B, B)] = wy
    wb[pl.ds(2 * B, B)] = wz
    a0 = z0 * (G * G)
    a1 = z1 * (G * G)
    b0 = y0 * G
    b1 = y1 * G
    idxb[pl.ds(0 * B, B)] = a0 + b0 + x0
    idxb[pl.ds(1 * B, B)] = a0 + b0 + x1
    idxb[pl.ds(2 * B, B)] = a0 + b1 + x0
    idxb[pl.ds(3 * B, B)] = a0 + b1 + x1
    idxb[pl.ds(4 * B, B)] = a1 + b0 + x0
    idxb[pl.ds(5 * B, B)] = a1 + b0 + x1
    idxb[pl.ds(6 * B, B)] = a1 + b1 + x0
    idxb[pl.ds(7 * B, B)] = a1 + b1 + x1


def _combine(rows, wb, outb):
    def point(p, pcarry):
        pv = jnp.full((B,), p, jnp.int32)
        wxp = plsc.load_gather(wb, [pv])
        wyp = plsc.load_gather(wb, [pv + B])
        wzp = plsc.load_gather(wb, [pv + 2 * B])
        for cg in range(C // 16):
            sl = pl.ds(cg * 16, 16)
            c000 = rows[0 * B + p, sl]
            c001 = rows[1 * B + p, sl]
            c010 = rows[2 * B + p, sl]
            c011 = rows[3 * B + p, sl]
            c100 = rows[4 * B + p, sl]
            c101 = rows[5 * B + p, sl]
            c110 = rows[6 * B + p, sl]
            c111 = rows[7 * B + p, sl]
            c00 = c000 + wxp * (c001 - c000)
            c01 = c010 + wxp * (c011 - c010)
            c10 = c100 + wxp * (c101 - c100)
            c11 = c110 + wxp * (c111 - c110)
            c0 = c00 + wyp * (c01 - c00)
            c1 = c10 + wyp * (c11 - c10)
            outb[p, pl.ds(C + cg * 16, 16)] = c0 + wzp * (c1 - c0)
        return pcarry

    lax.fori_loop(0, B, point, 0)


def _sc_sample(table, vxyz, pf):
    mesh = plsc.VectorSubcoreMesh(core_axis_name="c", subcore_axis_name="s")

    scratch = [pltpu.VMEM((3 * WPTS,), jnp.float32)]
    scratch += [pltpu.VMEM((8 * B,), jnp.int32) for _ in range(NB)]
    scratch += [pltpu.VMEM((3 * B,), jnp.float32) for _ in range(NB)]
    scratch += [pltpu.VMEM((8 * B, C), jnp.float32) for _ in range(NB)]
    scratch += [pltpu.VMEM((B, 2 * C), jnp.float32) for _ in range(NB)]
    scratch += [pltpu.SemaphoreType.DMA for _ in range(3 * NB)]

    @functools.partial(
        pl.kernel,
        out_type=jax.ShapeDtypeStruct((N, 2 * C), jnp.float32),
        mesh=mesh,
        compiler_params=pltpu.CompilerParams(needs_layout_passes=False),
        scratch_types=scratch,
    )
    def k(table_h, vxyz_h, pf_h, out_h, vbuf, *bufs):
        idxb = bufs[0:NB]
        wb = bufs[NB:2 * NB]
        rows = bufs[2 * NB:3 * NB]
        outb = bufs[3 * NB:4 * NB]
        gsem = bufs[4 * NB:5 * NB]
        psem = bufs[5 * NB:6 * NB]
        osem = bufs[6 * NB:7 * NB]

        wid = lax.axis_index("s") * NC + lax.axis_index("c")
        wbase = ((wid * ((N - WPTS) // 8)) // (NW - 1)) * 8

        pltpu.sync_copy(vxyz_h.at[pl.ds(wid * 3 * WPTS, 3 * WPTS)], vbuf)

        def fill_from(off, b):
            _fill_block(
                vbuf[pl.ds(0 * WPTS + off, B)],
                vbuf[pl.ds(1 * WPTS + off, B)],
                vbuf[pl.ds(2 * WPTS + off, B)],
                idxb[b],
                wb[b],
            )

        def gather_start(b):
            pltpu.async_copy(table_h.at[idxb[b]], rows[b], gsem[b])

        def gather_wait(b):
            pltpu.make_async_copy(table_h.at[idxb[b]], rows[b], gsem[b]).wait()

        def pf_start(i, b):
            pltpu.async_copy(
                pf_h.at[pl.ds(wbase + i * B, B), :],
                outb[b].at[:, pl.ds(0, C)],
                psem[b],
            )

        def pf_wait(i, b):
            pltpu.make_async_copy(
                pf_h.at[pl.ds(wbase + i * B, B), :],
                outb[b].at[:, pl.ds(0, C)],
                psem[b],
            ).wait()

        def out_start(i, b):
            pltpu.async_copy(
                outb[b], out_h.at[pl.ds(wbase + i * B, B), :], osem[b]
            )

        def out_wait(i, b):
            pltpu.make_async_copy(
                outb[b], out_h.at[pl.ds(wbase + i * B, B), :], osem[b]
            ).wait()

        for b in range(NB):
            fill_from(b * B, b)
            gather_start(b)

        def outer(ii, carry):
            for b in range(NB):
                i = ii * NB + b
                gather_wait(b)

                @pl.when(ii > 0)
                def _():
                    out_wait(i, b)

                pf_start(i, b)
                _combine(rows[b], wb[b], outb[b])

                @pl.when(ii < NBLK // NB - 1)
                def _():
                    fill_from((i + NB) * B, b)
                    gather_start(b)

                pf_wait(i, b)
                out_start(i, b)
            return carry

        lax.fori_loop(0, NBLK // NB, outer, 0)

        for b in range(NB):
            out_wait(NBLK - NB + b, b)

    return k(table, vxyz, pf)


def kernel(grid_batch_features, vertices, point_feat):
    grid = grid_batch_features[0].reshape(C, S)
    table = _transpose(grid)
    idxmat = jnp.asarray(
        np.add.outer(np.array(STARTS), np.arange(WPTS)), jnp.int32
    )
    vxyz = vertices[idxmat].transpose(0, 2, 1).reshape(NW * 3 * WPTS)
    return _sc_sample(table, vxyz, point_feat)
